# Initial kernel scaffold; baseline (speedup 1.0000x reference)
import jax
import jax.numpy as jnp
from jax.experimental import pallas as pl


def kernel(x, edge_index, W1, b1, W2, b2, W3, b3, B, Wn, Np):
    raise NotImplementedError("write your pallas kernel here")


# speedup vs baseline: 7.6392x; 7.6392x over previous
import functools

import jax
import jax.numpy as jnp
from jax import lax
from jax.experimental import pallas as pl
from jax.experimental.pallas import tpu as pltpu
from jax.experimental.pallas import tpu_sc as plsc

N = 10000
E = 160000
D = 256
DH = 128
NC = 2
NS = 16
CHA = 79
CHD = 40
NBLK = 1000
TRASH = N
NACC = 10016

_f32 = jnp.float32
_i32 = jnp.int32

_MESH = plsc.VectorSubcoreMesh(core_axis_name="c", subcore_axis_name="s")


@functools.partial(
    pl.kernel,
    out_type=jax.ShapeDtypeStruct((NC, NACC, 128), _f32),
    mesh=_MESH,
    scratch_types=[
        pltpu.VMEM((CHD, 128), _i32),
        pltpu.VMEM((128, 128), _f32),
        pltpu.VMEM((128, 128), _f32),
        pltpu.VMEM_SHARED((NACC, 128), _f32),
    ],
)
def _deg_kernel(dst_hbm, const_hbm, out_hbm, didx, ones_b, zero_b, acc):
    c = lax.axis_index("c")
    s = lax.axis_index("s")

    pltpu.sync_copy(const_hbm.at[0], ones_b)
    pltpu.sync_copy(const_hbm.at[1], zero_b)
    pltpu.sync_copy(dst_hbm.at[c, s], didx)

    @pl.when(s < NS - 1)
    def _():
        for k in range(5):
            pltpu.sync_copy(zero_b, acc.at[pl.ds(s * 640 + k * 128, 128)])

    @pl.when(s == NS - 1)
    def _():
        for k in range(3):
            pltpu.sync_copy(zero_b, acc.at[pl.ds(9600 + k * 128, 128)])
        pltpu.sync_copy(zero_b.at[pl.ds(0, 32)], acc.at[pl.ds(9984, 32)])

    plsc.subcore_barrier()

    def body(j, _):
        pltpu.sync_copy(ones_b, acc.at[didx.at[j]], add=True)
        return 0

    lax.fori_loop(0, CHD, body, 0)
    plsc.subcore_barrier()

    @pl.when(s < NS - 1)
    def _():
        pltpu.sync_copy(acc.at[pl.ds(s * 640, 640)], out_hbm.at[c, pl.ds(s * 640, 640)])

    @pl.when(s == NS - 1)
    def _():
        pltpu.sync_copy(acc.at[pl.ds(9600, 416)], out_hbm.at[c, pl.ds(9600, 416)])


@functools.partial(
    pl.kernel,
    out_type=jax.ShapeDtypeStruct((NC * N, DH), _f32),
    mesh=_MESH,
    scratch_types=[
        pltpu.VMEM((CHA, 128), _i32),
        pltpu.VMEM((CHA, 128), _i32),
        pltpu.VMEM((128, DH), _f32),
        pltpu.VMEM_SHARED((NACC, DH), _f32),
        pltpu.SemaphoreType.DMA,
    ],
)
def _agg_kernel(g_hbm, src_hbm, dst_hbm, out_hbm, sidx, didx, rows, acc, sem):
    c = lax.axis_index("c")
    s = lax.axis_index("s")

    pltpu.sync_copy(src_hbm.at[c, s], sidx)
    pltpu.sync_copy(dst_hbm.at[s], didx)

    @pl.when(s < NS - 1)
    def _():
        pltpu.sync_copy(g_hbm.at[pl.ds(c * N + s * 640, 640)], acc.at[pl.ds(s * 640, 640)])

    @pl.when(s == NS - 1)
    def _():
        pltpu.sync_copy(g_hbm.at[pl.ds(c * N + 9600, 400)], acc.at[pl.ds(9600, 400)])

    plsc.subcore_barrier()

    def body(j, _):
        pltpu.async_copy(g_hbm.at[sidx.at[j]], rows, sem).wait()
        pltpu.sync_copy(rows, acc.at[didx.at[j]], add=True)
        return 0

    lax.fori_loop(0, CHA, body, 0)
    plsc.subcore_barrier()

    @pl.when(s < NS - 1)
    def _():
        pltpu.sync_copy(acc.at[pl.ds(s * 640, 640)], out_hbm.at[pl.ds(c * N + s * 640, 640)])

    @pl.when(s == NS - 1)
    def _():
        pltpu.sync_copy(acc.at[pl.ds(9600, 400)], out_hbm.at[pl.ds(c * N + 9600, 400)])


def _dinv(deg_ref):
    return lax.rsqrt(deg_ref[0] + deg_ref[1] + 1.0)


def _pre_body(deg_ref, x_ref, w_ref, out_ref):
    dinv = _dinv(deg_ref)
    p = lax.dot_general(x_ref[...], w_ref[...], (((1,), (1,)), ((), ())),
                        precision=lax.Precision.HIGHEST,
                        preferred_element_type=_f32)
    out_ref[...] = (dinv * p)[None]


def _mid_body(deg_ref, s_ref, b_ref, w_ref, out_ref):
    dinv = _dinv(deg_ref)
    h0 = jnp.maximum(dinv * s_ref[0] + b_ref[0, :DH], 0.0)
    h1 = jnp.maximum(dinv * s_ref[1] + b_ref[0, DH:], 0.0)
    h = jnp.concatenate([h0, h1], axis=1)
    p = lax.dot_general(h, w_ref[...], (((1,), (1,)), ((), ())),
                        precision=lax.Precision.HIGHEST,
                        preferred_element_type=_f32)
    out_ref[...] = (dinv * p)[None]


def _post_body(deg_ref, s_ref, b_ref, cc_ref, out_ref):
    dinv = _dinv(deg_ref)
    h0 = jnp.maximum(dinv * s_ref[0] + b_ref[0, :DH], 0.0)
    h1 = jnp.maximum(dinv * s_ref[1] + b_ref[0, DH:], 0.0)
    out_ref[...] = jnp.concatenate([h0, h1], axis=1) + cc_ref[0, 0]


def _tc_pre(deg_pair, x, w):
    return pl.pallas_call(
        _pre_body,
        grid=(NC, N // NBLK),
        in_specs=[
            pl.BlockSpec((NC, NBLK, 1), lambda c, i: (0, i, 0)),
            pl.BlockSpec((NBLK, D), lambda c, i: (i, 0)),
            pl.BlockSpec((DH, D), lambda c, i: (c, 0)),
        ],
        out_specs=pl.BlockSpec((1, NBLK, DH), lambda c, i: (c, i, 0)),
        out_shape=jax.ShapeDtypeStruct((NC, N, DH), _f32),
    )(deg_pair, x, w).reshape(NC * N, DH)


def _tc_mid(deg_pair, s2, b, w):
    return pl.pallas_call(
        _mid_body,
        grid=(NC, N // NBLK),
        in_specs=[
            pl.BlockSpec((NC, NBLK, 1), lambda c, i: (0, i, 0)),
            pl.BlockSpec((NC, NBLK, DH), lambda c, i: (0, i, 0)),
            pl.BlockSpec((1, D), lambda c, i: (0, 0)),
            pl.BlockSpec((DH, D), lambda c, i: (c, 0)),
        ],
        out_specs=pl.BlockSpec((1, NBLK, DH), lambda c, i: (c, i, 0)),
        out_shape=jax.ShapeDtypeStruct((NC, N, DH), _f32),
    )(deg_pair, s2, b, w).reshape(NC * N, DH)


def _tc_post(deg_pair, s2, b, cc):
    return pl.pallas_call(
        _post_body,
        grid=(N // NBLK,),
        in_specs=[
            pl.BlockSpec((NC, NBLK, 1), lambda i: (0, i, 0)),
            pl.BlockSpec((NC, NBLK, DH), lambda i: (0, i, 0)),
            pl.BlockSpec((1, D), lambda i: (0, 0)),
            pl.BlockSpec((1, 1), lambda i: (0, 0)),
        ],
        out_specs=pl.BlockSpec((NBLK, D), lambda i: (i, 0)),
        out_shape=jax.ShapeDtypeStruct((N, D), _f32),
    )(deg_pair, s2, b, cc)


def kernel(x, edge_index, W1, b1, W2, b2, W3, b3, B, Wn, Np):
    src = edge_index[0].astype(_i32)
    dst = edge_index[1].astype(_i32)

    e_agg = NS * CHA * 128
    src_pa = jnp.concatenate([src, jnp.zeros((e_agg - E,), _i32)])
    dst_pa = jnp.concatenate([dst, jnp.full((e_agg - E,), TRASH, _i32)])
    src_agg = (src_pa.reshape(1, NS, CHA, 128)
               + (jnp.arange(NC, dtype=_i32) * N).reshape(NC, 1, 1, 1))
    dst_agg = dst_pa.reshape(NS, CHA, 128)

    e_deg = NC * NS * CHD * 128
    dst_pd = jnp.concatenate([dst, jnp.full((e_deg - E,), TRASH, _i32)])
    dst_deg = dst_pd.reshape(NC, NS, CHD, 128)

    deg_const = jnp.stack([jnp.ones((128, 128), _f32), jnp.zeros((128, 128), _f32)])
    degacc = _deg_kernel(dst_deg, deg_const)
    deg_pair = degacc[:, :N, :1]

    b1r = b1.reshape(1, D)
    b2r = b2.reshape(1, D)
    b3r = b3.reshape(1, D)
    cc = (jnp.asarray(B) * jnp.asarray(Wn) * jnp.asarray(Np) - N).astype(_f32).reshape(1, 1)

    g = _tc_pre(deg_pair, x, W1)
    s2 = _agg_kernel(g, src_agg, dst_agg).reshape(NC, N, DH)
    g = _tc_mid(deg_pair, s2, b1r, W2)
    s2 = _agg_kernel(g, src_agg, dst_agg).reshape(NC, N, DH)
    g = _tc_mid(deg_pair, s2, b2r, W3)
    s2 = _agg_kernel(g, src_agg, dst_agg).reshape(NC, N, DH)

    out = _tc_post(deg_pair, s2, b3r, cc)
    return out.reshape(2, 4, 1250, D)


# speedup vs baseline: 9.6853x; 1.2678x over previous
import functools

import jax
import jax.numpy as jnp
from jax import lax
from jax.experimental import pallas as pl
from jax.experimental.pallas import tpu as pltpu
from jax.experimental.pallas import tpu_sc as plsc

N = 10000
E = 160000
D = 256
DH = 128
NC = 2
NS = 16
CHA = 79
CHD = 40
NBLK = 1000
TRASH = N
NACC = 10016

_f32 = jnp.float32
_i32 = jnp.int32

_MESH = plsc.VectorSubcoreMesh(core_axis_name="c", subcore_axis_name="s")


@functools.partial(
    pl.kernel,
    out_type=jax.ShapeDtypeStruct((NC, NACC, 128), _f32),
    mesh=_MESH,
    scratch_types=[
        pltpu.VMEM((CHD, 128), _i32),
        pltpu.VMEM((128, 128), _f32),
        pltpu.VMEM((128, 128), _f32),
        pltpu.VMEM_SHARED((NACC, 128), _f32),
        pltpu.SemaphoreType.DMA,
    ],
)
def _deg_kernel(dst_hbm, const_hbm, out_hbm, didx, ones_b, zero_b, acc, ssem):
    c = lax.axis_index("c")
    s = lax.axis_index("s")

    pltpu.sync_copy(const_hbm.at[0], ones_b)
    pltpu.sync_copy(const_hbm.at[1], zero_b)
    pltpu.sync_copy(dst_hbm.at[c, s], didx)

    @pl.when(s < NS - 1)
    def _():
        for k in range(5):
            pltpu.sync_copy(zero_b, acc.at[pl.ds(s * 640 + k * 128, 128)])

    @pl.when(s == NS - 1)
    def _():
        for k in range(3):
            pltpu.sync_copy(zero_b, acc.at[pl.ds(9600 + k * 128, 128)])
        pltpu.sync_copy(zero_b.at[pl.ds(0, 32)], acc.at[pl.ds(9984, 32)])

    plsc.subcore_barrier()

    def body(j, _):
        pltpu.async_copy(ones_b, acc.at[didx.at[j]], ssem, add=True)
        return 0

    lax.fori_loop(0, CHD, body, 0)

    def drain(j, _):
        pltpu.make_async_copy(ones_b, acc.at[didx.at[0]], ssem).wait()
        return 0

    lax.fori_loop(0, CHD, drain, 0)
    plsc.subcore_barrier()

    @pl.when(s < NS - 1)
    def _():
        pltpu.sync_copy(acc.at[pl.ds(s * 640, 640)], out_hbm.at[c, pl.ds(s * 640, 640)])

    @pl.when(s == NS - 1)
    def _():
        pltpu.sync_copy(acc.at[pl.ds(9600, 416)], out_hbm.at[c, pl.ds(9600, 416)])


@functools.partial(
    pl.kernel,
    out_type=jax.ShapeDtypeStruct((NC * N, DH), _f32),
    mesh=_MESH,
    scratch_types=[
        pltpu.VMEM((CHA, 128), _i32),
        pltpu.VMEM((16, 128), _i32),
        pltpu.VMEM((128, DH), _f32),
        pltpu.VMEM((128, DH), _f32),
        pltpu.VMEM_SHARED((NACC, DH), _f32),
        pltpu.SemaphoreType.DMA,
        pltpu.SemaphoreType.DMA,
    ],
)
def _agg_kernel(g_hbm, src_hbm, dst_hbm, out_hbm, sidx, didxb, rows0, rows1, acc, sem0, sem1):
    c = lax.axis_index("c")
    s = lax.axis_index("s")

    pltpu.sync_copy(src_hbm.at[c, s], sidx)
    pltpu.async_copy(g_hbm.at[sidx.at[0]], rows0, sem0)

    @pl.when(s < NS - 1)
    def _():
        pltpu.sync_copy(g_hbm.at[pl.ds(c * N + s * 640, 640)], acc.at[pl.ds(s * 640, 640)])

    @pl.when(s == NS - 1)
    def _():
        pltpu.sync_copy(g_hbm.at[pl.ds(c * N + 9600, 400)], acc.at[pl.ds(9600, 400)])

    plsc.subcore_barrier()

    def _step(j, i, cur, csem, nxt, nsem):
        @pl.when(j < CHA - 1)
        def _():
            pltpu.async_copy(g_hbm.at[sidx.at[j + 1]], nxt, nsem)

        pltpu.make_async_copy(g_hbm.at[sidx.at[j]], cur, csem).wait()
        pltpu.sync_copy(cur, acc.at[didxb.at[i]], add=True)

    for base in range(0, CHA, 16):
        nch = min(16, CHA - base)
        pltpu.sync_copy(dst_hbm.at[s, pl.ds(base, nch)], didxb.at[pl.ds(0, nch)])

        def body(i, _, base=base):
            j = base + i

            @pl.when(j % 2 == 0)
            def _():
                _step(j, i, rows0, sem0, rows1, sem1)

            @pl.when(j % 2 == 1)
            def _():
                _step(j, i, rows1, sem1, rows0, sem0)

            return 0

        lax.fori_loop(0, nch, body, 0)
    plsc.subcore_barrier()

    @pl.when(s < NS - 1)
    def _():
        pltpu.sync_copy(acc.at[pl.ds(s * 640, 640)], out_hbm.at[pl.ds(c * N + s * 640, 640)])

    @pl.when(s == NS - 1)
    def _():
        pltpu.sync_copy(acc.at[pl.ds(9600, 400)], out_hbm.at[pl.ds(c * N + 9600, 400)])


def _dinv(deg_ref):
    return lax.rsqrt(deg_ref[0, :, :1] + deg_ref[1, :, :1] + 1.0)


def _pre_body(deg_ref, x_ref, w_ref, out_ref):
    dinv = _dinv(deg_ref)
    p = lax.dot_general(x_ref[...], w_ref[...], (((1,), (1,)), ((), ())),
                        precision=lax.Precision.HIGHEST,
                        preferred_element_type=_f32)
    out_ref[...] = (dinv * p)[None]


def _mid_body(deg_ref, s_ref, b_ref, w_ref, out_ref):
    dinv = _dinv(deg_ref)
    h0 = jnp.maximum(dinv * s_ref[0] + b_ref[0, :DH], 0.0)
    h1 = jnp.maximum(dinv * s_ref[1] + b_ref[0, DH:], 0.0)
    h = jnp.concatenate([h0, h1], axis=1)
    p = lax.dot_general(h, w_ref[...], (((1,), (1,)), ((), ())),
                        precision=lax.Precision.HIGHEST,
                        preferred_element_type=_f32)
    out_ref[...] = (dinv * p)[None]


def _post_body(deg_ref, s_ref, b_ref, cc_ref, out_ref):
    dinv = _dinv(deg_ref)
    h0 = jnp.maximum(dinv * s_ref[0] + b_ref[0, :DH], 0.0)
    h1 = jnp.maximum(dinv * s_ref[1] + b_ref[0, DH:], 0.0)
    out_ref[...] = jnp.concatenate([h0, h1], axis=1) + cc_ref[0, 0]


def _tc_pre(deg_pair, x, w):
    return pl.pallas_call(
        _pre_body,
        grid=(NC, N // NBLK),
        in_specs=[
            pl.BlockSpec((NC, NBLK, 128), lambda c, i: (0, i, 0)),
            pl.BlockSpec((NBLK, D), lambda c, i: (i, 0)),
            pl.BlockSpec((DH, D), lambda c, i: (c, 0)),
        ],
        out_specs=pl.BlockSpec((1, NBLK, DH), lambda c, i: (c, i, 0)),
        out_shape=jax.ShapeDtypeStruct((NC, N, DH), _f32),
    )(deg_pair, x, w).reshape(NC * N, DH)


def _tc_mid(deg_pair, s2, b, w):
    return pl.pallas_call(
        _mid_body,
        grid=(NC, N // NBLK),
        in_specs=[
            pl.BlockSpec((NC, NBLK, 128), lambda c, i: (0, i, 0)),
            pl.BlockSpec((NC, NBLK, DH), lambda c, i: (0, i, 0)),
            pl.BlockSpec((1, D), lambda c, i: (0, 0)),
            pl.BlockSpec((DH, D), lambda c, i: (c, 0)),
        ],
        out_specs=pl.BlockSpec((1, NBLK, DH), lambda c, i: (c, i, 0)),
        out_shape=jax.ShapeDtypeStruct((NC, N, DH), _f32),
    )(deg_pair, s2, b, w).reshape(NC * N, DH)


def _tc_post(deg_pair, s2, b, cc):
    return pl.pallas_call(
        _post_body,
        grid=(N // NBLK,),
        in_specs=[
            pl.BlockSpec((NC, NBLK, 128), lambda i: (0, i, 0)),
            pl.BlockSpec((NC, NBLK, DH), lambda i: (0, i, 0)),
            pl.BlockSpec((1, D), lambda i: (0, 0)),
            pl.BlockSpec((1, 1), lambda i: (0, 0)),
        ],
        out_specs=pl.BlockSpec((NBLK, D), lambda i: (i, 0)),
        out_shape=jax.ShapeDtypeStruct((N, D), _f32),
    )(deg_pair, s2, b, cc)


def kernel(x, edge_index, W1, b1, W2, b2, W3, b3, B, Wn, Np):
    src = edge_index[0].astype(_i32)
    dst = edge_index[1].astype(_i32)

    e_agg = NS * CHA * 128
    src_pa = jnp.concatenate([src, jnp.zeros((e_agg - E,), _i32)])
    dst_pa = jnp.concatenate([dst, jnp.full((e_agg - E,), TRASH, _i32)])
    src_agg = (src_pa.reshape(1, NS, CHA, 128)
               + (jnp.arange(NC, dtype=_i32) * N).reshape(NC, 1, 1, 1))
    dst_agg = dst_pa.reshape(NS, CHA, 128)

    e_deg = NC * NS * CHD * 128
    dst_pd = jnp.concatenate([dst, jnp.full((e_deg - E,), TRASH, _i32)])
    dst_deg = dst_pd.reshape(NC, NS, CHD, 128)

    deg_const = jnp.stack([jnp.ones((128, 128), _f32), jnp.zeros((128, 128), _f32)])
    degacc = _deg_kernel(dst_deg, deg_const)
    deg_pair = degacc

    b1r = b1.reshape(1, D)
    b2r = b2.reshape(1, D)
    b3r = b3.reshape(1, D)
    cc = (jnp.asarray(B) * jnp.asarray(Wn) * jnp.asarray(Np) - N).astype(_f32).reshape(1, 1)

    g = _tc_pre(deg_pair, x, W1)
    s2 = _agg_kernel(g, src_agg, dst_agg).reshape(NC, N, DH)
    g = _tc_mid(deg_pair, s2, b1r, W2)
    s2 = _agg_kernel(g, src_agg, dst_agg).reshape(NC, N, DH)
    g = _tc_mid(deg_pair, s2, b2r, W3)
    s2 = _agg_kernel(g, src_agg, dst_agg).reshape(NC, N, DH)

    out = _tc_post(deg_pair, s2, b3r, cc)
    return out.reshape(2, 4, 1250, D)


# speedup vs baseline: 9.8028x; 1.0121x over previous
import functools

import jax
import jax.numpy as jnp
from jax import lax
from jax.experimental import pallas as pl
from jax.experimental.pallas import tpu as pltpu
from jax.experimental.pallas import tpu_sc as plsc

N = 10000
E = 160000
D = 256
DH = 128
NC = 2
NS = 16
CHA = 79
CHD = 40
NBLK = 1000
TRASH = N
NACC = 10016

_f32 = jnp.float32
_i32 = jnp.int32

_MESH = plsc.VectorSubcoreMesh(core_axis_name="c", subcore_axis_name="s")


@functools.partial(
    pl.kernel,
    out_type=jax.ShapeDtypeStruct((NC, NACC, 128), _f32),
    mesh=_MESH,
    scratch_types=[
        pltpu.VMEM((CHD, 128), _i32),
        pltpu.VMEM((128, 128), _f32),
        pltpu.VMEM((128, 128), _f32),
        pltpu.VMEM_SHARED((NACC, 128), _f32),
        pltpu.SemaphoreType.DMA,
    ],
)
def _deg_kernel(dst_hbm, const_hbm, out_hbm, didx, ones_b, zero_b, acc, ssem):
    c = lax.axis_index("c")
    s = lax.axis_index("s")

    pltpu.sync_copy(const_hbm.at[0], ones_b)
    pltpu.sync_copy(const_hbm.at[1], zero_b)
    pltpu.sync_copy(dst_hbm.at[c, s], didx)

    @pl.when(s < NS - 1)
    def _():
        for k in range(5):
            pltpu.sync_copy(zero_b, acc.at[pl.ds(s * 640 + k * 128, 128)])

    @pl.when(s == NS - 1)
    def _():
        for k in range(3):
            pltpu.sync_copy(zero_b, acc.at[pl.ds(9600 + k * 128, 128)])
        pltpu.sync_copy(zero_b.at[pl.ds(0, 32)], acc.at[pl.ds(9984, 32)])

    plsc.subcore_barrier()

    def body(j, _):
        pltpu.async_copy(ones_b, acc.at[didx.at[j]], ssem, add=True)
        return 0

    lax.fori_loop(0, CHD, body, 0)

    def drain(j, _):
        pltpu.make_async_copy(ones_b, acc.at[didx.at[0]], ssem).wait()
        return 0

    lax.fori_loop(0, CHD, drain, 0)
    plsc.subcore_barrier()

    @pl.when(s < NS - 1)
    def _():
        pltpu.sync_copy(acc.at[pl.ds(s * 640, 640)], out_hbm.at[c, pl.ds(s * 640, 640)])

    @pl.when(s == NS - 1)
    def _():
        pltpu.sync_copy(acc.at[pl.ds(9600, 416)], out_hbm.at[c, pl.ds(9600, 416)])


@functools.partial(
    pl.kernel,
    out_type=jax.ShapeDtypeStruct((NC * N, DH), _f32),
    mesh=_MESH,
    scratch_types=[
        pltpu.VMEM((CHA, 128), _i32),
        pltpu.VMEM((16, 128), _i32),
        pltpu.VMEM((128, DH), _f32),
        pltpu.VMEM((128, DH), _f32),
        pltpu.VMEM_SHARED((NACC, DH), _f32),
        pltpu.SemaphoreType.DMA,
        pltpu.SemaphoreType.DMA,
        pltpu.SemaphoreType.DMA,
        pltpu.SemaphoreType.DMA,
    ],
)
def _agg_kernel(g_hbm, src_hbm, dst_hbm, out_hbm, sidx, didxb, rows0, rows1, acc,
                sem0, sem1, ssem0, ssem1):
    c = lax.axis_index("c")
    s = lax.axis_index("s")

    pltpu.sync_copy(src_hbm.at[c, s], sidx)
    pltpu.async_copy(g_hbm.at[sidx.at[0]], rows0, sem0)

    @pl.when(s < NS - 1)
    def _():
        pltpu.sync_copy(g_hbm.at[pl.ds(c * N + s * 640, 640)], acc.at[pl.ds(s * 640, 640)])

    @pl.when(s == NS - 1)
    def _():
        pltpu.sync_copy(g_hbm.at[pl.ds(c * N + 9600, 400)], acc.at[pl.ds(9600, 400)])

    plsc.subcore_barrier()

    def _step(j, i, cur, csem, cssem, nxt, nsem, nssem):
        @pl.when((i >= 1) & (j < CHA - 1))
        def _():
            pltpu.make_async_copy(nxt, acc.at[didxb.at[0]], nssem).wait()
            pltpu.async_copy(g_hbm.at[sidx.at[j + 1]], nxt, nsem)

        @pl.when((i >= 1) & (j >= CHA - 1))
        def _():
            pltpu.make_async_copy(nxt, acc.at[didxb.at[0]], nssem).wait()

        @pl.when((i < 1) & (j < CHA - 1))
        def _():
            pltpu.async_copy(g_hbm.at[sidx.at[j + 1]], nxt, nsem)

        pltpu.make_async_copy(g_hbm.at[sidx.at[j]], cur, csem).wait()
        pltpu.async_copy(cur, acc.at[didxb.at[i]], cssem, add=True)

    for base in range(0, CHA, 16):
        nch = min(16, CHA - base)
        if base > 0:
            pltpu.make_async_copy(rows1, acc.at[didxb.at[0]], ssem1).wait()
        pltpu.sync_copy(dst_hbm.at[s, pl.ds(base, nch)], didxb.at[pl.ds(0, nch)])

        def body(i, _, base=base):
            j = base + i

            @pl.when(j % 2 == 0)
            def _():
                _step(j, i, rows0, sem0, ssem0, rows1, sem1, ssem1)

            @pl.when(j % 2 == 1)
            def _():
                _step(j, i, rows1, sem1, ssem1, rows0, sem0, ssem0)

            return 0

        lax.fori_loop(0, nch, body, 0)

    pltpu.make_async_copy(rows0, acc.at[didxb.at[0]], ssem0).wait()
    plsc.subcore_barrier()

    @pl.when(s < NS - 1)
    def _():
        pltpu.sync_copy(acc.at[pl.ds(s * 640, 640)], out_hbm.at[pl.ds(c * N + s * 640, 640)])

    @pl.when(s == NS - 1)
    def _():
        pltpu.sync_copy(acc.at[pl.ds(9600, 400)], out_hbm.at[pl.ds(c * N + 9600, 400)])


def _dinv(deg_ref):
    return lax.rsqrt(deg_ref[0, :, :1] + deg_ref[1, :, :1] + 1.0)


def _pre_body(deg_ref, x_ref, w_ref, out_ref):
    dinv = _dinv(deg_ref)
    p = lax.dot_general(x_ref[...], w_ref[...], (((1,), (1,)), ((), ())),
                        precision=lax.Precision.HIGHEST,
                        preferred_element_type=_f32)
    out_ref[...] = (dinv * p)[None]


def _mid_body(deg_ref, s_ref, b_ref, w_ref, out_ref):
    dinv = _dinv(deg_ref)
    h0 = jnp.maximum(dinv * s_ref[0] + b_ref[0, :DH], 0.0)
    h1 = jnp.maximum(dinv * s_ref[1] + b_ref[0, DH:], 0.0)
    h = jnp.concatenate([h0, h1], axis=1)
    p = lax.dot_general(h, w_ref[...], (((1,), (1,)), ((), ())),
                        precision=lax.Precision.HIGHEST,
                        preferred_element_type=_f32)
    out_ref[...] = (dinv * p)[None]


def _post_body(deg_ref, s_ref, b_ref, cc_ref, out_ref):
    dinv = _dinv(deg_ref)
    h0 = jnp.maximum(dinv * s_ref[0] + b_ref[0, :DH], 0.0)
    h1 = jnp.maximum(dinv * s_ref[1] + b_ref[0, DH:], 0.0)
    out_ref[...] = jnp.concatenate([h0, h1], axis=1) + cc_ref[0, 0]


def _tc_pre(deg_pair, x, w):
    return pl.pallas_call(
        _pre_body,
        grid=(NC, N // NBLK),
        in_specs=[
            pl.BlockSpec((NC, NBLK, 128), lambda c, i: (0, i, 0)),
            pl.BlockSpec((NBLK, D), lambda c, i: (i, 0)),
            pl.BlockSpec((DH, D), lambda c, i: (c, 0)),
        ],
        out_specs=pl.BlockSpec((1, NBLK, DH), lambda c, i: (c, i, 0)),
        out_shape=jax.ShapeDtypeStruct((NC, N, DH), _f32),
    )(deg_pair, x, w).reshape(NC * N, DH)


def _tc_mid(deg_pair, s2, b, w):
    return pl.pallas_call(
        _mid_body,
        grid=(NC, N // NBLK),
        in_specs=[
            pl.BlockSpec((NC, NBLK, 128), lambda c, i: (0, i, 0)),
            pl.BlockSpec((NC, NBLK, DH), lambda c, i: (0, i, 0)),
            pl.BlockSpec((1, D), lambda c, i: (0, 0)),
            pl.BlockSpec((DH, D), lambda c, i: (c, 0)),
        ],
        out_specs=pl.BlockSpec((1, NBLK, DH), lambda c, i: (c, i, 0)),
        out_shape=jax.ShapeDtypeStruct((NC, N, DH), _f32),
    )(deg_pair, s2, b, w).reshape(NC * N, DH)


def _tc_post(deg_pair, s2, b, cc):
    return pl.pallas_call(
        _post_body,
        grid=(N // NBLK,),
        in_specs=[
            pl.BlockSpec((NC, NBLK, 128), lambda i: (0, i, 0)),
            pl.BlockSpec((NC, NBLK, DH), lambda i: (0, i, 0)),
            pl.BlockSpec((1, D), lambda i: (0, 0)),
            pl.BlockSpec((1, 1), lambda i: (0, 0)),
        ],
        out_specs=pl.BlockSpec((NBLK, D), lambda i: (i, 0)),
        out_shape=jax.ShapeDtypeStruct((N, D), _f32),
    )(deg_pair, s2, b, cc)


def kernel(x, edge_index, W1, b1, W2, b2, W3, b3, B, Wn, Np):
    src = edge_index[0].astype(_i32)
    dst = edge_index[1].astype(_i32)

    e_agg = NS * CHA * 128
    src_pa = jnp.concatenate([src, jnp.zeros((e_agg - E,), _i32)])
    dst_pa = jnp.concatenate([dst, jnp.full((e_agg - E,), TRASH, _i32)])
    src_agg = (src_pa.reshape(1, NS, CHA, 128)
               + (jnp.arange(NC, dtype=_i32) * N).reshape(NC, 1, 1, 1))
    dst_agg = dst_pa.reshape(NS, CHA, 128)

    e_deg = NC * NS * CHD * 128
    dst_pd = jnp.concatenate([dst, jnp.full((e_deg - E,), TRASH, _i32)])
    dst_deg = dst_pd.reshape(NC, NS, CHD, 128)

    deg_const = jnp.stack([jnp.ones((128, 128), _f32), jnp.zeros((128, 128), _f32)])
    degacc = _deg_kernel(dst_deg, deg_const)
    deg_pair = degacc

    b1r = b1.reshape(1, D)
    b2r = b2.reshape(1, D)
    b3r = b3.reshape(1, D)
    cc = (jnp.asarray(B) * jnp.asarray(Wn) * jnp.asarray(Np) - N).astype(_f32).reshape(1, 1)

    g = _tc_pre(deg_pair, x, W1)
    s2 = _agg_kernel(g, src_agg, dst_agg).reshape(NC, N, DH)
    g = _tc_mid(deg_pair, s2, b1r, W2)
    s2 = _agg_kernel(g, src_agg, dst_agg).reshape(NC, N, DH)
    g = _tc_mid(deg_pair, s2, b2r, W3)
    s2 = _agg_kernel(g, src_agg, dst_agg).reshape(NC, N, DH)

    out = _tc_post(deg_pair, s2, b3r, cc)
    return out.reshape(2, 4, 1250, D)
